# baseline (speedup 1.0000x reference)
import jax
import jax.numpy as jnp
from jax import lax
from jax.experimental import pallas as pl
from jax.experimental.pallas import tpu as pltpu
from jax.experimental.pallas import tpu_sc as plsc

NC = 2
NS = 16
NW = NC * NS
CH = 128
N_IDXU = 8
N_PAIRU = 5


def _sc_mesh():
    return plsc.VectorSubcoreMesh(
        core_axis_name="c", subcore_axis_name="s",
        num_cores=NC, num_subcores=NS)


def _pair_tables(a, b):
    V, D = a.shape
    R = 4096

    def body(at_ref, bt_ref, o_ref):
        eye = jnp.eye(D, dtype=jnp.float32)
        dn = (((0,), (0,)), ((), ()))
        ta = jax.lax.dot_general(at_ref[...], eye, dn,
                                 preferred_element_type=jnp.float32)
        tb = jax.lax.dot_general(bt_ref[...], eye, dn,
                                 preferred_element_type=jnp.float32)
        o_ref[...] = jnp.concatenate([ta, tb], axis=1).astype(jnp.bfloat16)

    return pl.pallas_call(
        body,
        grid=((V + R - 1) // R,),
        in_specs=[
            pl.BlockSpec((D, R), lambda i: (0, i)),
            pl.BlockSpec((D, R), lambda i: (0, i)),
        ],
        out_specs=pl.BlockSpec((R, 2 * D), lambda i: (i, 0)),
        out_shape=jax.ShapeDtypeStruct((V, 2 * D), jnp.bfloat16),
        compiler_params=pltpu.CompilerParams(
            dimension_semantics=("arbitrary",),
            fuse_transposed_lhs_in_matmul=True),
    )(a.T, b.T)


def _sc_gather_user(idx2d, user_tab, nch, B):
    rows_per_idx = B // CH

    def body(idx_hbm, ut_hbm, out_hbm, idx_v, buf_v, sem):
        wid = lax.axis_index("s") * NC + lax.axis_index("c")
        for r in range(N_IDXU):
            pltpu.sync_copy(
                idx_hbm.at[pl.ds(r * rows_per_idx + wid * nch, nch)],
                idx_v.at[r])

        jobs = [
            (0, [(0, 0, 128)]),
            (1, [(1, 0, 64)]),
            (2, [(1, 64, 64)]),
            (3, [(2, 0, 64)]),
            (4, [(2, 64, 64)]),
            (5, [(3, 0, 64)]),
            (6, [(3, 64, 64)]),
            (7, [(4, 0, 64), (4, 64, 64)]),
        ]

        def do_chunk(ci, carry):
            for wave in (jobs[:4], jobs[4:]):
                cps = [
                    pltpu.async_copy(
                        ut_hbm.at[idx_v.at[r, ci]],
                        buf_v.at[bi], sem)
                    for bi, (r, _) in enumerate(wave)
                ]
                for cp in cps:
                    cp.wait()
                for bi, (_, writes) in enumerate(wave):
                    for (p, off, w) in writes:
                        src = buf_v.at[bi] if w == 128 else \
                            buf_v.at[bi, :, pl.ds(0, 64)]
                        pltpu.sync_copy(
                            src, out_hbm.at[p, wid, ci, :, pl.ds(off, w)])
            return carry

        lax.fori_loop(0, nch, do_chunk, 0)

    fn = pl.kernel(
        body,
        out_type=jax.ShapeDtypeStruct((N_PAIRU, NW, nch, CH, 128),
                                      jnp.bfloat16),
        mesh=_sc_mesh(),
        scratch_types=[
            pltpu.VMEM((N_IDXU, nch, CH), jnp.int32),
            pltpu.VMEM((4, CH, 128), jnp.bfloat16),
            pltpu.SemaphoreType.DMA,
        ],
        compiler_params=pltpu.CompilerParams(use_tc_tiling_on_sc=False),
    )
    return fn(idx2d, user_tab)


def _sc_gather_item(item2d, item_tab, nch, B):
    def body(idx_hbm, it_hbm, out_hbm, idx_v, buf_v, sem):
        wid = lax.axis_index("s") * NC + lax.axis_index("c")
        pltpu.sync_copy(idx_hbm.at[pl.ds(wid * nch, nch)], idx_v)

        def do_chunk(ci, carry):
            pltpu.async_copy(
                it_hbm.at[idx_v.at[ci]],
                buf_v, sem).wait()
            pltpu.sync_copy(buf_v, out_hbm.at[wid, ci])
            return carry

        lax.fori_loop(0, nch, do_chunk, 0)

    fn = pl.kernel(
        body,
        out_type=jax.ShapeDtypeStruct((NW, nch, CH, 128), jnp.bfloat16),
        mesh=_sc_mesh(),
        scratch_types=[
            pltpu.VMEM((nch, CH), jnp.int32),
            pltpu.VMEM((CH, 128), jnp.bfloat16),
            pltpu.SemaphoreType.DMA,
        ],
        compiler_params=pltpu.CompilerParams(use_tc_tiling_on_sc=False),
    )
    return fn(item2d, item_tab)


def _mlp_body(pu_ref, pi_ref, w1_ref, b1_ref, w2_ref, b2_ref, w3_ref,
              b3_ref, wo_ref, bo_ref, o_ref):
    acc = jnp.dot(pi_ref[...], w1_ref[1], preferred_element_type=jnp.float32)
    for j, s in enumerate((0, 2, 3, 4, 5)):
        acc = acc + jnp.dot(pu_ref[j], w1_ref[s],
                            preferred_element_type=jnp.float32)
    h1 = jnp.maximum(acc + b1_ref[...], 0.0)
    h2 = jnp.maximum(
        jnp.dot(h1, w2_ref[...], preferred_element_type=jnp.float32)
        + b2_ref[...], 0.0)
    h3 = jnp.maximum(
        jnp.dot(h2, w3_ref[...], preferred_element_type=jnp.float32)
        + b3_ref[...], 0.0)
    gmf = (pu_ref[0][:, 64:].astype(jnp.float32)
           * pi_ref[...][:, 64:].astype(jnp.float32))
    wo = wo_ref[...]
    out = (jnp.sum(gmf * wo[:, :64], axis=1)
           + jnp.sum(h3 * wo[:, 64:], axis=1))
    o_ref[...] = out + bo_ref[0]


def _mlp(pairs_u, pair_i, w1p, b1, w2, b2, w3, b3, wo, bo, blk):
    B = pair_i.shape[0]
    return pl.pallas_call(
        _mlp_body,
        grid=(B // blk,),
        in_specs=[
            pl.BlockSpec((N_PAIRU, blk, 128), lambda i: (0, i, 0)),
            pl.BlockSpec((blk, 128), lambda i: (i, 0)),
            pl.BlockSpec((6, 128, 256), lambda i: (0, 0, 0)),
            pl.BlockSpec((1, 256), lambda i: (0, 0)),
            pl.BlockSpec((256, 128), lambda i: (0, 0)),
            pl.BlockSpec((1, 128), lambda i: (0, 0)),
            pl.BlockSpec((128, 64), lambda i: (0, 0)),
            pl.BlockSpec((1, 64), lambda i: (0, 0)),
            pl.BlockSpec((1, 128), lambda i: (0, 0)),
            pl.BlockSpec(memory_space=pltpu.SMEM),
        ],
        out_specs=pl.BlockSpec((blk,), lambda i: (i,)),
        out_shape=jax.ShapeDtypeStruct((B,), jnp.float32),
        compiler_params=pltpu.CompilerParams(
            dimension_semantics=("arbitrary",)),
    )(pairs_u, pair_i, w1p, b1, w2, b2, w3, b3, wo, bo)


def kernel(user_id, item_id, context_id, mlp_user, mlp_item, gmf_user,
           gmf_item, W1, b1, W2, b2, W3, b3, Wout, bout):
    B = user_id.shape[0]
    user_id = user_id.astype(jnp.int32)
    item_id = item_id.astype(jnp.int32)
    ctx_t = context_id.astype(jnp.int32).T

    user_tab = _pair_tables(mlp_user, gmf_user)
    idx2d = jnp.concatenate(
        [user_id, ctx_t.reshape(-1)]).reshape(-1, CH)
    nch = B // NW // CH

    pairs_u5 = _sc_gather_user(idx2d, user_tab, nch, B)
    pairs_u = pairs_u5.reshape(N_PAIRU, B, 128)

    item_tab = _pair_tables(mlp_item, gmf_item)
    pair_i4 = _sc_gather_item(item_id.reshape(-1, CH), item_tab, nch, B)
    pair_i = pair_i4.reshape(B, 128)

    blocks = W1.reshape(9, 64, 256)
    z = jnp.zeros((64, 256), W1.dtype)
    w1p = jnp.stack([
        jnp.concatenate([blocks[0], z]),
        jnp.concatenate([blocks[1], z]),
        jnp.concatenate([blocks[2], blocks[3]]),
        jnp.concatenate([blocks[4], blocks[5]]),
        jnp.concatenate([blocks[6], blocks[7]]),
        jnp.concatenate([blocks[8], z]),
    ]).astype(jnp.bfloat16)
    return _mlp(pairs_u, pair_i, w1p, b1.reshape(1, 256), W2,
                b2.reshape(1, 128), W3, b3.reshape(1, 64),
                Wout.reshape(1, 128), bout, blk=1024)


# speedup vs baseline: 2.3561x; 2.3561x over previous
import jax
import jax.numpy as jnp
from jax import lax
from jax.experimental import pallas as pl
from jax.experimental.pallas import tpu as pltpu
from jax.experimental.pallas import tpu_sc as plsc

NC = 2
NS = 16
NW = NC * NS
CH = 128
N_IDXU = 8
N_PAIRU = 5


def _sc_mesh():
    return plsc.VectorSubcoreMesh(
        core_axis_name="c", subcore_axis_name="s",
        num_cores=NC, num_subcores=NS)


def _pair_tables(a, b):
    V, D = a.shape
    R = 4096

    def body(at_ref, bt_ref, o_ref):
        eye = jnp.eye(D, dtype=jnp.float32)
        dn = (((0,), (0,)), ((), ()))
        ta = jax.lax.dot_general(at_ref[...], eye, dn,
                                 preferred_element_type=jnp.float32)
        tb = jax.lax.dot_general(bt_ref[...], eye, dn,
                                 preferred_element_type=jnp.float32)
        o_ref[...] = jnp.concatenate([ta, tb], axis=1)

    return pl.pallas_call(
        body,
        grid=((V + R - 1) // R,),
        in_specs=[
            pl.BlockSpec((D, R), lambda i: (0, i)),
            pl.BlockSpec((D, R), lambda i: (0, i)),
        ],
        out_specs=pl.BlockSpec((R, 2 * D), lambda i: (i, 0)),
        out_shape=jax.ShapeDtypeStruct((V, 2 * D), jnp.float32),
        compiler_params=pltpu.CompilerParams(
            dimension_semantics=("arbitrary",),
            fuse_transposed_lhs_in_matmul=True),
    )(a.T, b.T)


def _sc_gather_user(idx2d, user_tab, nch, B):
    rows_per_idx = B // CH

    def body(idx_hbm, ut_hbm, out_hbm, idx_v, buf_v, sem):
        wid = lax.axis_index("s") * NC + lax.axis_index("c")
        for r in range(N_IDXU):
            pltpu.sync_copy(
                idx_hbm.at[pl.ds(r * rows_per_idx + wid * nch, nch)],
                idx_v.at[r])

        jobs = [
            (0, [(0, 0, 128)]),
            (1, [(1, 0, 64)]),
            (2, [(1, 64, 64)]),
            (3, [(2, 0, 64)]),
            (4, [(2, 64, 64)]),
            (5, [(3, 0, 64)]),
            (6, [(3, 64, 64)]),
            (7, [(4, 0, 64), (4, 64, 64)]),
        ]

        def do_chunk(ci, carry):
            for wave in (jobs[:4], jobs[4:]):
                cps = [
                    pltpu.async_copy(
                        ut_hbm.at[idx_v.at[r, ci]],
                        buf_v.at[bi], sem)
                    for bi, (r, _) in enumerate(wave)
                ]
                for cp in cps:
                    cp.wait()
                for bi, (_, writes) in enumerate(wave):
                    for (p, off, w) in writes:
                        src = buf_v.at[bi] if w == 128 else \
                            buf_v.at[bi, :, pl.ds(0, 64)]
                        pltpu.sync_copy(
                            src, out_hbm.at[p, wid, ci, :, pl.ds(off, w)])
            return carry

        lax.fori_loop(0, nch, do_chunk, 0)

    fn = pl.kernel(
        body,
        out_type=jax.ShapeDtypeStruct((N_PAIRU, NW, nch, CH, 128),
                                      jnp.float32),
        mesh=_sc_mesh(),
        scratch_types=[
            pltpu.VMEM((N_IDXU, nch, CH), jnp.int32),
            pltpu.VMEM((4, CH, 128), jnp.float32),
            pltpu.SemaphoreType.DMA,
        ],
        compiler_params=pltpu.CompilerParams(use_tc_tiling_on_sc=False),
    )
    return fn(idx2d, user_tab)


def _sc_gather_item(item2d, item_tab, nch, B):
    def body(idx_hbm, it_hbm, out_hbm, idx_v, buf_v, sem):
        wid = lax.axis_index("s") * NC + lax.axis_index("c")
        pltpu.sync_copy(idx_hbm.at[pl.ds(wid * nch, nch)], idx_v)

        def do_chunk(ci, carry):
            pltpu.async_copy(
                it_hbm.at[idx_v.at[ci]],
                buf_v, sem).wait()
            pltpu.sync_copy(buf_v, out_hbm.at[wid, ci])
            return carry

        lax.fori_loop(0, nch, do_chunk, 0)

    fn = pl.kernel(
        body,
        out_type=jax.ShapeDtypeStruct((NW, nch, CH, 128), jnp.float32),
        mesh=_sc_mesh(),
        scratch_types=[
            pltpu.VMEM((nch, CH), jnp.int32),
            pltpu.VMEM((CH, 128), jnp.float32),
            pltpu.SemaphoreType.DMA,
        ],
        compiler_params=pltpu.CompilerParams(use_tc_tiling_on_sc=False),
    )
    return fn(item2d, item_tab)


def _mlp_body(pu_ref, pi_ref, w1_ref, b1_ref, w2_ref, b2_ref, w3_ref,
              b3_ref, wo_ref, bo_ref, o_ref):
    acc = jnp.dot(pi_ref[...], w1_ref[1], preferred_element_type=jnp.float32)
    for j, s in enumerate((0, 2, 3, 4, 5)):
        acc = acc + jnp.dot(pu_ref[j], w1_ref[s],
                            preferred_element_type=jnp.float32)
    h1 = jnp.maximum(acc + b1_ref[...], 0.0)
    h2 = jnp.maximum(
        jnp.dot(h1, w2_ref[...], preferred_element_type=jnp.float32)
        + b2_ref[...], 0.0)
    h3 = jnp.maximum(
        jnp.dot(h2, w3_ref[...], preferred_element_type=jnp.float32)
        + b3_ref[...], 0.0)
    gmf = (pu_ref[0][:, 64:].astype(jnp.float32)
           * pi_ref[...][:, 64:].astype(jnp.float32))
    wo = wo_ref[...]
    out = (jnp.sum(gmf * wo[:, :64], axis=1)
           + jnp.sum(h3 * wo[:, 64:], axis=1))
    o_ref[...] = out + bo_ref[0]


def _mlp(pairs_u, pair_i, w1p, b1, w2, b2, w3, b3, wo, bo, blk):
    B = pair_i.shape[0]
    return pl.pallas_call(
        _mlp_body,
        grid=(B // blk,),
        in_specs=[
            pl.BlockSpec((N_PAIRU, blk, 128), lambda i: (0, i, 0)),
            pl.BlockSpec((blk, 128), lambda i: (i, 0)),
            pl.BlockSpec((6, 128, 256), lambda i: (0, 0, 0)),
            pl.BlockSpec((1, 256), lambda i: (0, 0)),
            pl.BlockSpec((256, 128), lambda i: (0, 0)),
            pl.BlockSpec((1, 128), lambda i: (0, 0)),
            pl.BlockSpec((128, 64), lambda i: (0, 0)),
            pl.BlockSpec((1, 64), lambda i: (0, 0)),
            pl.BlockSpec((1, 128), lambda i: (0, 0)),
            pl.BlockSpec(memory_space=pltpu.SMEM),
        ],
        out_specs=pl.BlockSpec((blk,), lambda i: (i,)),
        out_shape=jax.ShapeDtypeStruct((B,), jnp.float32),
        compiler_params=pltpu.CompilerParams(
            dimension_semantics=("arbitrary",)),
    )(pairs_u, pair_i, w1p, b1, w2, b2, w3, b3, wo, bo)


def kernel(user_id, item_id, context_id, mlp_user, mlp_item, gmf_user,
           gmf_item, W1, b1, W2, b2, W3, b3, Wout, bout):
    B = user_id.shape[0]
    user_id = user_id.astype(jnp.int32)
    item_id = item_id.astype(jnp.int32)
    ctx_t = context_id.astype(jnp.int32).T

    user_tab = _pair_tables(mlp_user, gmf_user)
    idx2d = jnp.concatenate(
        [user_id, ctx_t.reshape(-1)]).reshape(-1, CH)
    nch = B // NW // CH

    pairs_u5 = _sc_gather_user(idx2d, user_tab, nch, B)
    pairs_u = pairs_u5.reshape(N_PAIRU, B, 128)

    item_tab = _pair_tables(mlp_item, gmf_item)
    pair_i4 = _sc_gather_item(item_id.reshape(-1, CH), item_tab, nch, B)
    pair_i = pair_i4.reshape(B, 128)

    blocks = W1.reshape(9, 64, 256)
    z = jnp.zeros((64, 256), W1.dtype)
    w1p = jnp.stack([
        jnp.concatenate([blocks[0], z]),
        jnp.concatenate([blocks[1], z]),
        jnp.concatenate([blocks[2], blocks[3]]),
        jnp.concatenate([blocks[4], blocks[5]]),
        jnp.concatenate([blocks[6], blocks[7]]),
        jnp.concatenate([blocks[8], z]),
    ])
    return _mlp(pairs_u, pair_i, w1p, b1.reshape(1, 256), W2,
                b2.reshape(1, 128), W3, b3.reshape(1, 64),
                Wout.reshape(1, 128), bout, blk=1024)
